# baseline (speedup 1.0000x reference)
import functools

import jax
import jax.numpy as jnp
from jax import lax
from jax.experimental import pallas as pl
from jax.experimental.pallas import tpu as pltpu
from jax.experimental.pallas import tpu_sc as plsc

VOCAB = 1000000
D = 64
DF = 128
BATCH = 4096
SEQ = 200
NC, NS, L = 2, 16, 16
NW = NC * NS
BW = BATCH // NW
NBUF = 4
NGROUP = SEQ // NBUF


def _embed_kernel(fat_hbm, xt_hbm, out_hbm, idx_v, *bufs):
    rows = bufs[:NBUF]
    comp = bufs[NBUF:2 * NBUF]
    gsem = bufs[2 * NBUF:3 * NBUF]
    osem = bufs[3 * NBUF:4 * NBUF]

    wid = lax.axis_index("s") * NC + lax.axis_index("c")
    b0 = wid * BW

    pltpu.sync_copy(xt_hbm.at[:, pl.ds(b0, BW)], idx_v)

    def gather_start(b, l):
        src = fat_hbm.at[idx_v.at[l]]
        pltpu.make_async_copy(src, rows[b], gsem[b]).start()

    for b in range(NBUF):
        gather_start(b, b)

    lane = lax.iota(jnp.int32, 16)

    def group(g, _):
        for b in range(NBUF):
            l = g * NBUF + b
            pltpu.make_async_copy(fat_hbm.at[idx_v.at[l]], rows[b],
                                  gsem[b]).wait()

            @plsc.parallel_loop(0, D, step=1)
            def transpose_scale(d):
                col = jnp.full((16,), 0, jnp.int32) + d
                for tb in range(BW // 16):
                    tok = tb * 16 + lane
                    v = plsc.load_gather(rows[b], [tok, col])
                    comp[b][d, pl.ds(tb * 16, 16)] = v * 8.0

            dst = out_hbm.at[l, :, pl.ds(b0, BW)]
            pltpu.make_async_copy(comp[b], dst, osem[b]).start()

        for b in range(NBUF):
            l = g * NBUF + b
            dst = out_hbm.at[l, :, pl.ds(b0, BW)]
            pltpu.make_async_copy(comp[b], dst, osem[b]).wait()

            @pl.when(g + 1 < NGROUP)
            def _():
                gather_start(b, (g + 1) * NBUF + b)

        return 0

    lax.fori_loop(0, NGROUP, group, 0)


@jax.jit
def _embed(fat, xt):
    mesh = plsc.VectorSubcoreMesh(core_axis_name="c", subcore_axis_name="s")
    f = functools.partial(
        pl.kernel,
        out_type=jax.ShapeDtypeStruct((SEQ, D, BATCH), jnp.float32),
        mesh=mesh,
        scratch_types=(
            [pltpu.VMEM((SEQ, BW), jnp.int32)]
            + [pltpu.VMEM((BW, DF), jnp.float32) for _ in range(NBUF)]
            + [pltpu.VMEM((D, BW), jnp.float32) for _ in range(NBUF)]
            + [pltpu.SemaphoreType.DMA for _ in range(2 * NBUF)]
        ),
        compiler_params=pltpu.CompilerParams(use_tc_tiling_on_sc=True,
                                             needs_layout_passes=False),
    )(_embed_kernel)
    return f(fat, xt)


def kernel(x, table):
    fat = jnp.pad(table, ((0, 0), (0, DF - D)))
    xt = x.T.astype(jnp.int32)
    out = _embed(fat, xt)
    return jnp.transpose(out, (2, 0, 1))
